# baseline (speedup 1.0000x reference)
import functools
import math

import jax
import jax.numpy as jnp
from jax import lax
from jax.experimental import pallas as pl
from jax.experimental.pallas import tpu as pltpu
from jax.experimental.pallas import tpu_sc as plsc

_K = 16
_T = 512


def _ln(x, g, b, eps=1e-5):
    m = jnp.mean(x, axis=-1, keepdims=True)
    v = jnp.mean((x - m) ** 2, axis=-1, keepdims=True)
    return (x - m) / jnp.sqrt(v + eps) * g + b


def _gelu(x):
    return 0.5 * x * (1.0 + jax.lax.erf(x / math.sqrt(2.0)))


def _dist_idx_kernel(ctr_tile_ref, ctr_t_ref, x_tile_ref, x_full_ref,
                     g_ref, b_ref, idx_ref, xn_ref, tqs_ref):
    b = pl.program_id(0)
    t = pl.program_id(1)
    ct = ctr_tile_ref[0]
    xcT = ctr_t_ref[0]
    G = xcT.shape[1]
    sqt = jnp.sum(ct * ct, axis=1, keepdims=True)
    sqf = jnp.sum(xcT * xcT, axis=0, keepdims=True)
    cross = jax.lax.dot_general(ct, xcT, (((1,), (0,)), ((), ())),
                                preferred_element_type=jnp.float32)
    d2 = sqt + sqf - 2.0 * cross

    iota = jax.lax.broadcasted_iota(jnp.int32, d2.shape, 1)
    dm = d2
    cols = []
    for _ in range(_K):
        rowmin = jnp.min(dm, axis=1, keepdims=True)
        hit = dm <= rowmin
        cand = jnp.where(hit, iota, G)
        cols.append(jnp.min(cand, axis=1, keepdims=True))
        dm = jnp.where(hit, jnp.float32(jnp.inf), dm)
    idx_ref[0] = jnp.transpose(
        jnp.concatenate(cols, axis=1) + b * G)

    g = g_ref[...]
    bb = b_ref[...]
    xn = _ln(x_tile_ref[0], g, bb)
    xn_ref[0] = xn

    sel = jnp.where(jnp.isinf(dm), 1.0, 0.0)
    colsum = jnp.transpose(jnp.sum(sel, axis=0, keepdims=True))
    xnf = _ln(x_full_ref[0], g, bb)
    tqpart = jnp.sum(colsum * xnf * xnf)

    @pl.when(jnp.logical_and(b == 0, t == 0))
    def _():
        tqs_ref[0] = 0.0

    tqs_ref[0] += tqpart


def _sc_gather_sum(xn_tab, idx_t, B, G):
    P = xn_tab.shape[0]
    C = xn_tab.shape[1]
    NW = 32
    ppw = P // NW
    CH = 16
    nch = ppw // CH
    wpb = NW // B
    mesh = plsc.VectorSubcoreMesh(core_axis_name="c", subcore_axis_name="s")

    @functools.partial(
        pl.kernel, mesh=mesh,
        out_type=jax.ShapeDtypeStruct((P, C), jnp.float32),
        scratch_types=[
            pltpu.VMEM((_K * ppw,), jnp.int32),
            pltpu.VMEM((CH * _K,), jnp.int32),
            pltpu.VMEM((CH * _K, C), jnp.float32),
            pltpu.VMEM((CH, C), jnp.float32),
            pltpu.SemaphoreType.DMA,
        ],
    )
    def k(tab_hbm, idxt_hbm, out_hbm,
          idx_all, idx_ch, rows_v, acc_v, sem):
        wid = lax.axis_index("s") * 2 + lax.axis_index("c")
        b = wid // wpb
        g0 = (wid % wpb) * ppw
        for kk in range(_K):
            pltpu.sync_copy(
                idxt_hbm.at[pl.ds(b * _K * G + kk * G + g0, ppw)],
                idx_all.at[pl.ds(kk * ppw, ppw)])

        def chunk_body(c, carry):
            base = wid * ppw + c * CH
            for kk in range(_K):
                idx_ch[pl.ds(kk * CH, CH)] = idx_all[pl.ds(kk * ppw + c * CH, CH)]
            pltpu.async_copy(tab_hbm.at[idx_ch.at[pl.ds(0, 128)]],
                             rows_v.at[pl.ds(0, 128)], sem).wait()
            pltpu.async_copy(tab_hbm.at[idx_ch.at[pl.ds(128, 128)]],
                             rows_v.at[pl.ds(128, 128)], sem).wait()
            for p in range(CH):
                for d0 in range(C // 16):
                    a = rows_v[p, pl.ds(d0 * 16, 16)]
                    for kk in range(1, _K):
                        a = a + rows_v[kk * CH + p, pl.ds(d0 * 16, 16)]
                    acc_v[p, pl.ds(d0 * 16, 16)] = a
            pltpu.sync_copy(acc_v, out_hbm.at[pl.ds(base, CH)])
            return carry

        lax.fori_loop(0, nch, chunk_body, 0)

    return k(xn_tab, idx_t)


def _sums_kernel(t1_ref, xn_ref, sums_ref):
    b = pl.program_id(0)
    t = pl.program_id(1)
    t1 = t1_ref[0]
    xn = xn_ref[0]
    psy = jnp.sum(t1) - _K * jnp.sum(xn)
    psy2 = _K * jnp.sum(xn * xn) - 2.0 * jnp.sum(xn * t1)

    @pl.when(jnp.logical_and(b == 0, t == 0))
    def _():
        sums_ref[0] = 0.0
        sums_ref[1] = 0.0

    sums_ref[0] += psy
    sums_ref[1] += psy2


def _mlp_kernel(t1_ref, x_tile_ref, xcls_ref, scal_ref,
                g1_ref, b1_ref, g2_ref, b2_ref, al_ref, be_ref,
                aw1_ref, ab1_ref, aw2_ref, ab2_ref,
                mw1_ref, mb1_ref, mw2_ref, mb2_ref,
                out_ref, outc_ref):
    b = pl.program_id(0)
    t = pl.program_id(1)
    stdinv = scal_ref[0]
    g1 = g1_ref[...]
    bb1 = b1_ref[...]
    g2 = g2_ref[...]
    bb2 = b2_ref[...]

    xt = x_tile_ref[0]
    xn = _ln(xt, g1, bb1)
    s = t1_ref[0] * (1.0 / _K)
    e1 = (s - xn) * stdinv
    enh = jnp.concatenate([e1, xn], axis=1)
    enh = enh * al_ref[...] + be_ref[...]
    h = _gelu(jnp.dot(enh, aw1_ref[...],
                      preferred_element_type=jnp.float32) + ab1_ref[...])
    a = jnp.dot(h, aw2_ref[...],
                preferred_element_type=jnp.float32) + ab2_ref[...]
    y = xt + a
    y2 = _ln(y, g2, bb2)
    m = _gelu(jnp.dot(y2, mw1_ref[...],
                      preferred_element_type=jnp.float32) + mb1_ref[...])
    out_ref[0] = y + jnp.dot(m, mw2_ref[...],
                             preferred_element_type=jnp.float32) + mb2_ref[...]

    @pl.when(jnp.logical_and(b == 0, t == 0))
    def _():
        xc = xcls_ref[...]
        c0 = xc + _ln(xc, g1, bb1)
        y2c = _ln(c0, g2, bb2)
        mc = _gelu(jnp.dot(y2c, mw1_ref[...],
                           preferred_element_type=jnp.float32) + mb1_ref[...])
        outc_ref[...] = c0 + jnp.dot(mc, mw2_ref[...],
                                     preferred_element_type=jnp.float32) + mb2_ref[...]


def kernel(center, x, affine_alpha, affine_beta, ln1_g, ln1_b,
           attn_w1, attn_b1, attn_w2, attn_b2, ln2_g, ln2_b,
           mlp_w1, mlp_b1, mlp_w2, mlp_b2):
    B, Np1, C = x.shape
    G = Np1 - 1
    nT = G // _T
    H = mlp_w1.shape[1]

    x_cls = x[:, 0, :]
    x_rest = x[:, 1:, :]
    ctr_t = jnp.transpose(center, (0, 2, 1))
    g1 = ln1_g.reshape(1, C)
    b1 = ln1_b.reshape(1, C)
    g2 = ln2_g.reshape(1, C)
    b2 = ln2_b.reshape(1, C)

    idx, xn_tab, tqs = pl.pallas_call(
        _dist_idx_kernel,
        grid=(B, nT),
        in_specs=[
            pl.BlockSpec((1, _T, 3), lambda b, t: (b, t, 0)),
            pl.BlockSpec((1, 3, G), lambda b, t: (b, 0, 0)),
            pl.BlockSpec((1, _T, C), lambda b, t: (b, t, 0)),
            pl.BlockSpec((1, G, C), lambda b, t: (b, 0, 0)),
            pl.BlockSpec((1, C), lambda b, t: (0, 0)),
            pl.BlockSpec((1, C), lambda b, t: (0, 0)),
        ],
        out_specs=[
            pl.BlockSpec((1, _K, _T), lambda b, t: (b, 0, t)),
            pl.BlockSpec((1, _T, C), lambda b, t: (b, t, 0)),
            pl.BlockSpec(memory_space=pltpu.SMEM),
        ],
        out_shape=[
            jax.ShapeDtypeStruct((B, _K, G), jnp.int32),
            jax.ShapeDtypeStruct((B, G, C), jnp.float32),
            jax.ShapeDtypeStruct((1,), jnp.float32),
        ],
    )(center, ctr_t, x_rest, x_rest, g1, b1)

    t1 = _sc_gather_sum(xn_tab.reshape(B * G, C),
                        idx.reshape(B * _K * G), B, G).reshape(B, G, C)

    sums = pl.pallas_call(
        _sums_kernel,
        grid=(B, nT),
        in_specs=[
            pl.BlockSpec((1, _T, C), lambda b, t: (b, t, 0)),
            pl.BlockSpec((1, _T, C), lambda b, t: (b, t, 0)),
        ],
        out_specs=pl.BlockSpec(memory_space=pltpu.SMEM),
        out_shape=jax.ShapeDtypeStruct((2,), jnp.float32),
    )(t1, xn_tab)

    M = B * G * _K * C
    var = (sums[1] + tqs[0] - sums[0] * sums[0] / M) / (M - 1)
    stdinv = (1.0 / (jnp.sqrt(var) + 1e-05)).reshape(1)

    out_rest, out_cls = pl.pallas_call(
        _mlp_kernel,
        grid=(B, nT),
        in_specs=[
            pl.BlockSpec((1, _T, C), lambda b, t: (b, t, 0)),
            pl.BlockSpec((1, _T, C), lambda b, t: (b, t, 0)),
            pl.BlockSpec((B, C), lambda b, t: (0, 0)),
            pl.BlockSpec(memory_space=pltpu.SMEM),
            pl.BlockSpec((1, C), lambda b, t: (0, 0)),
            pl.BlockSpec((1, C), lambda b, t: (0, 0)),
            pl.BlockSpec((1, C), lambda b, t: (0, 0)),
            pl.BlockSpec((1, C), lambda b, t: (0, 0)),
            pl.BlockSpec((1, 2 * C), lambda b, t: (0, 0)),
            pl.BlockSpec((1, 2 * C), lambda b, t: (0, 0)),
            pl.BlockSpec((2 * C, C), lambda b, t: (0, 0)),
            pl.BlockSpec((1, C), lambda b, t: (0, 0)),
            pl.BlockSpec((C, C), lambda b, t: (0, 0)),
            pl.BlockSpec((1, C), lambda b, t: (0, 0)),
            pl.BlockSpec((C, H), lambda b, t: (0, 0)),
            pl.BlockSpec((1, H), lambda b, t: (0, 0)),
            pl.BlockSpec((H, C), lambda b, t: (0, 0)),
            pl.BlockSpec((1, C), lambda b, t: (0, 0)),
        ],
        out_specs=[
            pl.BlockSpec((1, _T, C), lambda b, t: (b, t, 0)),
            pl.BlockSpec((B, C), lambda b, t: (0, 0)),
        ],
        out_shape=[
            jax.ShapeDtypeStruct((B, G, C), jnp.float32),
            jax.ShapeDtypeStruct((B, C), jnp.float32),
        ],
    )(t1, x_rest, x_cls, stdinv, g1, b1, g2, b2,
      affine_alpha.reshape(1, 2 * C), affine_beta.reshape(1, 2 * C),
      attn_w1, attn_b1.reshape(1, C), attn_w2, attn_b2.reshape(1, C),
      mlp_w1, mlp_b1.reshape(1, H), mlp_w2, mlp_b2.reshape(1, C))

    return jnp.concatenate([out_cls[:, None, :], out_rest], axis=1)


# speedup vs baseline: 1.0205x; 1.0205x over previous
import functools
import math

import jax
import jax.numpy as jnp
from jax import lax
from jax.experimental import pallas as pl
from jax.experimental.pallas import tpu as pltpu
from jax.experimental.pallas import tpu_sc as plsc

_K = 16
_T = 512


def _ln(x, g, b, eps=1e-5):
    m = jnp.mean(x, axis=-1, keepdims=True)
    v = jnp.mean((x - m) ** 2, axis=-1, keepdims=True)
    return (x - m) / jnp.sqrt(v + eps) * g + b


def _gelu(x):
    return 0.5 * x * (1.0 + jax.lax.erf(x / math.sqrt(2.0)))


def _dist_idx_kernel(ctr_tile_ref, ctr_t_ref, x_tile_ref, x_full_ref,
                     g_ref, b_ref, idx_ref, xn_ref, tqs_ref):
    b = pl.program_id(0)
    t = pl.program_id(1)
    ct = ctr_tile_ref[0]
    xcT = ctr_t_ref[0]
    G = xcT.shape[1]
    sqt = jnp.sum(ct * ct, axis=1, keepdims=True)
    sqf = jnp.sum(xcT * xcT, axis=0, keepdims=True)
    cross = jax.lax.dot_general(ct, xcT, (((1,), (0,)), ((), ())),
                                preferred_element_type=jnp.float32)
    d2 = sqt + sqf - 2.0 * cross

    iota = jax.lax.broadcasted_iota(jnp.int32, d2.shape, 1)
    dm = d2
    cols = []
    for _ in range(_K):
        rowmin = jnp.min(dm, axis=1, keepdims=True)
        hit = dm <= rowmin
        cand = jnp.where(hit, iota, G)
        cols.append(jnp.min(cand, axis=1, keepdims=True))
        dm = jnp.where(hit, jnp.float32(jnp.inf), dm)
    idx_ref[0] = jnp.transpose(
        jnp.concatenate(cols, axis=1) + b * G)

    g = g_ref[...]
    bb = b_ref[...]
    xn = _ln(x_tile_ref[0], g, bb)
    xn_ref[0] = xn

    sel = jnp.where(jnp.isinf(dm), 1.0, 0.0)
    colsum = jnp.transpose(jnp.sum(sel, axis=0, keepdims=True))
    xnf = _ln(x_full_ref[0], g, bb)
    tqpart = jnp.sum(colsum * xnf * xnf)

    @pl.when(jnp.logical_and(b == 0, t == 0))
    def _():
        tqs_ref[0] = 0.0

    tqs_ref[0] += tqpart


def _sc_gather_sum(xn_tab, idx_t, B, G):
    P = xn_tab.shape[0]
    C = xn_tab.shape[1]
    NW = 32
    ppw = P // NW
    CH = 16
    nch = ppw // CH
    wpb = NW // B
    mesh = plsc.VectorSubcoreMesh(core_axis_name="c", subcore_axis_name="s")

    @functools.partial(
        pl.kernel, mesh=mesh,
        out_type=jax.ShapeDtypeStruct((P, C), jnp.float32),
        scratch_types=[
            pltpu.VMEM((_K * ppw,), jnp.int32),
            pltpu.VMEM((2, CH * _K), jnp.int32),
            pltpu.VMEM((2, CH * _K, C), jnp.float32),
            pltpu.VMEM((CH, C), jnp.float32),
            pltpu.SemaphoreType.DMA,
        ],
    )
    def k(tab_hbm, idxt_hbm, out_hbm,
          idx_all, idx_ch, rows_v, acc_v, sem):
        wid = lax.axis_index("s") * 2 + lax.axis_index("c")
        b = wid // wpb
        g0 = (wid % wpb) * ppw
        for kk in range(_K):
            pltpu.sync_copy(
                idxt_hbm.at[pl.ds(b * _K * G + kk * G + g0, ppw)],
                idx_all.at[pl.ds(kk * ppw, ppw)])

        def fire(c, pb):
            for kk in range(_K):
                idx_ch[pb, pl.ds(kk * CH, CH)] = \
                    idx_all[pl.ds(kk * ppw + c * CH, CH)]
            pltpu.async_copy(tab_hbm.at[idx_ch.at[pb, pl.ds(0, 128)]],
                             rows_v.at[pb, pl.ds(0, 128)], sem)
            pltpu.async_copy(tab_hbm.at[idx_ch.at[pb, pl.ds(128, 128)]],
                             rows_v.at[pb, pl.ds(128, 128)], sem)

        fire(0, 0)

        def chunk_body(c, carry):
            pb = lax.rem(c, 2)
            base = wid * ppw + c * CH

            @pl.when(c + 1 < nch)
            def _():
                fire(c + 1, 1 - pb)

            pltpu.make_async_copy(tab_hbm.at[pl.ds(0, 128)],
                                  rows_v.at[pb, pl.ds(0, 128)], sem).wait()
            pltpu.make_async_copy(tab_hbm.at[pl.ds(0, 128)],
                                  rows_v.at[pb, pl.ds(128, 128)], sem).wait()
            for p in range(CH):
                for d0 in range(C // 16):
                    a = rows_v[pb, p, pl.ds(d0 * 16, 16)]
                    for kk in range(1, _K):
                        a = a + rows_v[pb, kk * CH + p, pl.ds(d0 * 16, 16)]
                    acc_v[p, pl.ds(d0 * 16, 16)] = a
            pltpu.sync_copy(acc_v, out_hbm.at[pl.ds(base, CH)])
            return carry

        lax.fori_loop(0, nch, chunk_body, 0)

    return k(xn_tab, idx_t)


def _sums_kernel(t1_ref, xn_ref, sums_ref):
    b = pl.program_id(0)
    t = pl.program_id(1)
    t1 = t1_ref[0]
    xn = xn_ref[0]
    psy = jnp.sum(t1) - _K * jnp.sum(xn)
    psy2 = _K * jnp.sum(xn * xn) - 2.0 * jnp.sum(xn * t1)

    @pl.when(jnp.logical_and(b == 0, t == 0))
    def _():
        sums_ref[0] = 0.0
        sums_ref[1] = 0.0

    sums_ref[0] += psy
    sums_ref[1] += psy2


def _mlp_kernel(t1_ref, x_tile_ref, xcls_ref, scal_ref,
                g1_ref, b1_ref, g2_ref, b2_ref, al_ref, be_ref,
                aw1_ref, ab1_ref, aw2_ref, ab2_ref,
                mw1_ref, mb1_ref, mw2_ref, mb2_ref,
                out_ref, outc_ref):
    b = pl.program_id(0)
    t = pl.program_id(1)
    stdinv = scal_ref[0]
    g1 = g1_ref[...]
    bb1 = b1_ref[...]
    g2 = g2_ref[...]
    bb2 = b2_ref[...]

    xt = x_tile_ref[0]
    xn = _ln(xt, g1, bb1)
    s = t1_ref[0] * (1.0 / _K)
    e1 = (s - xn) * stdinv
    enh = jnp.concatenate([e1, xn], axis=1)
    enh = enh * al_ref[...] + be_ref[...]
    h = _gelu(jnp.dot(enh, aw1_ref[...],
                      preferred_element_type=jnp.float32) + ab1_ref[...])
    a = jnp.dot(h, aw2_ref[...],
                preferred_element_type=jnp.float32) + ab2_ref[...]
    y = xt + a
    y2 = _ln(y, g2, bb2)
    m = _gelu(jnp.dot(y2, mw1_ref[...],
                      preferred_element_type=jnp.float32) + mb1_ref[...])
    out_ref[0] = y + jnp.dot(m, mw2_ref[...],
                             preferred_element_type=jnp.float32) + mb2_ref[...]

    @pl.when(jnp.logical_and(b == 0, t == 0))
    def _():
        xc = xcls_ref[...]
        c0 = xc + _ln(xc, g1, bb1)
        y2c = _ln(c0, g2, bb2)
        mc = _gelu(jnp.dot(y2c, mw1_ref[...],
                           preferred_element_type=jnp.float32) + mb1_ref[...])
        outc_ref[...] = c0 + jnp.dot(mc, mw2_ref[...],
                                     preferred_element_type=jnp.float32) + mb2_ref[...]


def kernel(center, x, affine_alpha, affine_beta, ln1_g, ln1_b,
           attn_w1, attn_b1, attn_w2, attn_b2, ln2_g, ln2_b,
           mlp_w1, mlp_b1, mlp_w2, mlp_b2):
    B, Np1, C = x.shape
    G = Np1 - 1
    nT = G // _T
    H = mlp_w1.shape[1]

    x_cls = x[:, 0, :]
    x_rest = x[:, 1:, :]
    ctr_t = jnp.transpose(center, (0, 2, 1))
    g1 = ln1_g.reshape(1, C)
    b1 = ln1_b.reshape(1, C)
    g2 = ln2_g.reshape(1, C)
    b2 = ln2_b.reshape(1, C)

    idx, xn_tab, tqs = pl.pallas_call(
        _dist_idx_kernel,
        grid=(B, nT),
        in_specs=[
            pl.BlockSpec((1, _T, 3), lambda b, t: (b, t, 0)),
            pl.BlockSpec((1, 3, G), lambda b, t: (b, 0, 0)),
            pl.BlockSpec((1, _T, C), lambda b, t: (b, t, 0)),
            pl.BlockSpec((1, G, C), lambda b, t: (b, 0, 0)),
            pl.BlockSpec((1, C), lambda b, t: (0, 0)),
            pl.BlockSpec((1, C), lambda b, t: (0, 0)),
        ],
        out_specs=[
            pl.BlockSpec((1, _K, _T), lambda b, t: (b, 0, t)),
            pl.BlockSpec((1, _T, C), lambda b, t: (b, t, 0)),
            pl.BlockSpec(memory_space=pltpu.SMEM),
        ],
        out_shape=[
            jax.ShapeDtypeStruct((B, _K, G), jnp.int32),
            jax.ShapeDtypeStruct((B, G, C), jnp.float32),
            jax.ShapeDtypeStruct((1,), jnp.float32),
        ],
    )(center, ctr_t, x_rest, x_rest, g1, b1)

    t1 = _sc_gather_sum(xn_tab.reshape(B * G, C),
                        idx.reshape(B * _K * G), B, G).reshape(B, G, C)

    sums = pl.pallas_call(
        _sums_kernel,
        grid=(B, nT),
        in_specs=[
            pl.BlockSpec((1, _T, C), lambda b, t: (b, t, 0)),
            pl.BlockSpec((1, _T, C), lambda b, t: (b, t, 0)),
        ],
        out_specs=pl.BlockSpec(memory_space=pltpu.SMEM),
        out_shape=jax.ShapeDtypeStruct((2,), jnp.float32),
    )(t1, xn_tab)

    M = B * G * _K * C
    var = (sums[1] + tqs[0] - sums[0] * sums[0] / M) / (M - 1)
    stdinv = (1.0 / (jnp.sqrt(var) + 1e-05)).reshape(1)

    out_rest, out_cls = pl.pallas_call(
        _mlp_kernel,
        grid=(B, nT),
        in_specs=[
            pl.BlockSpec((1, _T, C), lambda b, t: (b, t, 0)),
            pl.BlockSpec((1, _T, C), lambda b, t: (b, t, 0)),
            pl.BlockSpec((B, C), lambda b, t: (0, 0)),
            pl.BlockSpec(memory_space=pltpu.SMEM),
            pl.BlockSpec((1, C), lambda b, t: (0, 0)),
            pl.BlockSpec((1, C), lambda b, t: (0, 0)),
            pl.BlockSpec((1, C), lambda b, t: (0, 0)),
            pl.BlockSpec((1, C), lambda b, t: (0, 0)),
            pl.BlockSpec((1, 2 * C), lambda b, t: (0, 0)),
            pl.BlockSpec((1, 2 * C), lambda b, t: (0, 0)),
            pl.BlockSpec((2 * C, C), lambda b, t: (0, 0)),
            pl.BlockSpec((1, C), lambda b, t: (0, 0)),
            pl.BlockSpec((C, C), lambda b, t: (0, 0)),
            pl.BlockSpec((1, C), lambda b, t: (0, 0)),
            pl.BlockSpec((C, H), lambda b, t: (0, 0)),
            pl.BlockSpec((1, H), lambda b, t: (0, 0)),
            pl.BlockSpec((H, C), lambda b, t: (0, 0)),
            pl.BlockSpec((1, C), lambda b, t: (0, 0)),
        ],
        out_specs=[
            pl.BlockSpec((1, _T, C), lambda b, t: (b, t, 0)),
            pl.BlockSpec((B, C), lambda b, t: (0, 0)),
        ],
        out_shape=[
            jax.ShapeDtypeStruct((B, G, C), jnp.float32),
            jax.ShapeDtypeStruct((B, C), jnp.float32),
        ],
    )(t1, x_rest, x_cls, stdinv, g1, b1, g2, b2,
      affine_alpha.reshape(1, 2 * C), affine_beta.reshape(1, 2 * C),
      attn_w1, attn_b1.reshape(1, C), attn_w2, attn_b2.reshape(1, C),
      mlp_w1, mlp_b1.reshape(1, H), mlp_w2, mlp_b2.reshape(1, C))

    return jnp.concatenate([out_cls[:, None, :], out_rest], axis=1)


# speedup vs baseline: 1.0293x; 1.0086x over previous
import functools
import math

import jax
import jax.numpy as jnp
from jax import lax
from jax.experimental import pallas as pl
from jax.experimental.pallas import tpu as pltpu
from jax.experimental.pallas import tpu_sc as plsc

_K = 16
_T = 512


def _ln(x, g, b, eps=1e-5):
    m = jnp.mean(x, axis=-1, keepdims=True)
    v = jnp.mean((x - m) ** 2, axis=-1, keepdims=True)
    return (x - m) / jnp.sqrt(v + eps) * g + b


def _gelu(x):
    return 0.5 * x * (1.0 + jax.lax.erf(x / math.sqrt(2.0)))


def _dist_idx_kernel(ctr_tile_ref, ctr_t_ref, x_tile_ref, x_full_ref,
                     g_ref, b_ref, idx_ref, xn_ref, tqs_ref):
    b = pl.program_id(0)
    t = pl.program_id(1)
    ct = ctr_tile_ref[0]
    xcT = ctr_t_ref[0]
    G = xcT.shape[1]
    sqt = jnp.sum(ct * ct, axis=1, keepdims=True)
    sqf = jnp.sum(xcT * xcT, axis=0, keepdims=True)
    cross = jax.lax.dot_general(ct, xcT, (((1,), (0,)), ((), ())),
                                preferred_element_type=jnp.float32)
    d2 = sqt + sqf - 2.0 * cross

    iota = jax.lax.broadcasted_iota(jnp.int32, d2.shape, 1)
    dm = d2
    cols = []
    for _ in range(_K):
        rowmin = jnp.min(dm, axis=1, keepdims=True)
        hit = dm <= rowmin
        cand = jnp.where(hit, iota, G)
        cols.append(jnp.min(cand, axis=1, keepdims=True))
        dm = jnp.where(hit, jnp.float32(jnp.inf), dm)
    idx_ref[0] = jnp.transpose(
        jnp.concatenate(cols, axis=1) + b * G)

    g = g_ref[...]
    bb = b_ref[...]
    xn = _ln(x_tile_ref[0], g, bb)
    xn_ref[0] = xn

    sel = jnp.where(jnp.isinf(dm), 1.0, 0.0)
    colsum = jnp.transpose(jnp.sum(sel, axis=0, keepdims=True))
    xnf = _ln(x_full_ref[0], g, bb)
    tqpart = jnp.sum(colsum * xnf * xnf)

    @pl.when(jnp.logical_and(b == 0, t == 0))
    def _():
        tqs_ref[0] = 0.0

    tqs_ref[0] += tqpart


def _sc_gather_sum(xn_tab, idx_t, B, G):
    P = xn_tab.shape[0]
    C = xn_tab.shape[1]
    NW = 32
    ppw = P // NW
    CH = 16
    nch = ppw // CH
    wpb = NW // B
    assert (P // 2) % (2 * G) == 0 or P == 2 * 2 * G
    mesh = plsc.VectorSubcoreMesh(core_axis_name="c", subcore_axis_name="s")

    @functools.partial(
        pl.kernel, mesh=mesh,
        out_type=jax.ShapeDtypeStruct((P, C), jnp.float32),
        scratch_types=[
            pltpu.VMEM((_K * ppw,), jnp.int32),
            pltpu.VMEM((2, CH * _K), jnp.int32),
            pltpu.VMEM((2, CH * _K, C), jnp.float32),
            pltpu.VMEM((CH, C), jnp.float32),
            pltpu.VMEM_SHARED((P // 2, C), jnp.float32),
            pltpu.SemaphoreType.DMA,
        ],
    )
    def k(tab_hbm, idxt_hbm, out_hbm,
          idx_all, idx_ch, rows_v, acc_v, tab_sp, sem):
        cc = lax.axis_index("c")
        wid = cc * 16 + lax.axis_index("s")
        b = wid // wpb
        g0 = (wid % wpb) * ppw
        sp0 = cc * (P // 2)

        if True:
            @pl.when(lax.axis_index("s") == 0)
            def _():
                pltpu.sync_copy(tab_hbm.at[pl.ds(sp0, P // 2)], tab_sp)

            for kk in range(_K):
                pltpu.sync_copy(
                    idxt_hbm.at[pl.ds(b * _K * G + kk * G + g0, ppw)],
                    idx_all.at[pl.ds(kk * ppw, ppw)])
            plsc.subcore_barrier()

            def fire(c, pb):
                for kk in range(_K):
                    idx_ch[pb, pl.ds(kk * CH, CH)] = \
                        idx_all[pl.ds(kk * ppw + c * CH, CH)] - sp0
                pltpu.async_copy(tab_sp.at[idx_ch.at[pb, pl.ds(0, 128)]],
                                 rows_v.at[pb, pl.ds(0, 128)], sem)
                pltpu.async_copy(tab_sp.at[idx_ch.at[pb, pl.ds(128, 128)]],
                                 rows_v.at[pb, pl.ds(128, 128)], sem)

            fire(0, 0)

            def chunk_body(c, carry):
                pb = lax.rem(c, 2)
                base = wid * ppw + c * CH

                @pl.when(c + 1 < nch)
                def _():
                    fire(c + 1, 1 - pb)

                pltpu.make_async_copy(tab_hbm.at[pl.ds(0, 128)],
                                      rows_v.at[pb, pl.ds(0, 128)], sem).wait()
                pltpu.make_async_copy(tab_hbm.at[pl.ds(0, 128)],
                                      rows_v.at[pb, pl.ds(128, 128)], sem).wait()
                for p in range(CH):
                    for d0 in range(C // 16):
                        a = rows_v[pb, p, pl.ds(d0 * 16, 16)]
                        for kk in range(1, _K):
                            a = a + rows_v[pb, kk * CH + p, pl.ds(d0 * 16, 16)]
                        acc_v[p, pl.ds(d0 * 16, 16)] = a
                pltpu.sync_copy(acc_v, out_hbm.at[pl.ds(base, CH)])
                return carry

            lax.fori_loop(0, nch, chunk_body, 0)

    return k(xn_tab, idx_t)


def _sums_kernel(t1_ref, xn_ref, sums_ref):
    b = pl.program_id(0)
    t = pl.program_id(1)
    t1 = t1_ref[0]
    xn = xn_ref[0]
    psy = jnp.sum(t1) - _K * jnp.sum(xn)
    psy2 = _K * jnp.sum(xn * xn) - 2.0 * jnp.sum(xn * t1)

    @pl.when(jnp.logical_and(b == 0, t == 0))
    def _():
        sums_ref[0] = 0.0
        sums_ref[1] = 0.0

    sums_ref[0] += psy
    sums_ref[1] += psy2


def _mlp_kernel(t1_ref, x_tile_ref, xcls_ref, scal_ref,
                g1_ref, b1_ref, g2_ref, b2_ref, al_ref, be_ref,
                aw1_ref, ab1_ref, aw2_ref, ab2_ref,
                mw1_ref, mb1_ref, mw2_ref, mb2_ref,
                out_ref, outc_ref):
    b = pl.program_id(0)
    t = pl.program_id(1)
    stdinv = scal_ref[0]
    g1 = g1_ref[...]
    bb1 = b1_ref[...]
    g2 = g2_ref[...]
    bb2 = b2_ref[...]

    xt = x_tile_ref[0]
    xn = _ln(xt, g1, bb1)
    s = t1_ref[0] * (1.0 / _K)
    e1 = (s - xn) * stdinv
    enh = jnp.concatenate([e1, xn], axis=1)
    enh = enh * al_ref[...] + be_ref[...]
    h = _gelu(jnp.dot(enh, aw1_ref[...],
                      preferred_element_type=jnp.float32) + ab1_ref[...])
    a = jnp.dot(h, aw2_ref[...],
                preferred_element_type=jnp.float32) + ab2_ref[...]
    y = xt + a
    y2 = _ln(y, g2, bb2)
    m = _gelu(jnp.dot(y2, mw1_ref[...],
                      preferred_element_type=jnp.float32) + mb1_ref[...])
    out_ref[0] = y + jnp.dot(m, mw2_ref[...],
                             preferred_element_type=jnp.float32) + mb2_ref[...]

    @pl.when(jnp.logical_and(b == 0, t == 0))
    def _():
        xc = xcls_ref[...]
        c0 = xc + _ln(xc, g1, bb1)
        y2c = _ln(c0, g2, bb2)
        mc = _gelu(jnp.dot(y2c, mw1_ref[...],
                           preferred_element_type=jnp.float32) + mb1_ref[...])
        outc_ref[...] = c0 + jnp.dot(mc, mw2_ref[...],
                                     preferred_element_type=jnp.float32) + mb2_ref[...]


def kernel(center, x, affine_alpha, affine_beta, ln1_g, ln1_b,
           attn_w1, attn_b1, attn_w2, attn_b2, ln2_g, ln2_b,
           mlp_w1, mlp_b1, mlp_w2, mlp_b2):
    B, Np1, C = x.shape
    G = Np1 - 1
    nT = G // _T
    H = mlp_w1.shape[1]

    x_cls = x[:, 0, :]
    x_rest = x[:, 1:, :]
    ctr_t = jnp.transpose(center, (0, 2, 1))
    g1 = ln1_g.reshape(1, C)
    b1 = ln1_b.reshape(1, C)
    g2 = ln2_g.reshape(1, C)
    b2 = ln2_b.reshape(1, C)

    idx, xn_tab, tqs = pl.pallas_call(
        _dist_idx_kernel,
        grid=(B, nT),
        in_specs=[
            pl.BlockSpec((1, _T, 3), lambda b, t: (b, t, 0)),
            pl.BlockSpec((1, 3, G), lambda b, t: (b, 0, 0)),
            pl.BlockSpec((1, _T, C), lambda b, t: (b, t, 0)),
            pl.BlockSpec((1, G, C), lambda b, t: (b, 0, 0)),
            pl.BlockSpec((1, C), lambda b, t: (0, 0)),
            pl.BlockSpec((1, C), lambda b, t: (0, 0)),
        ],
        out_specs=[
            pl.BlockSpec((1, _K, _T), lambda b, t: (b, 0, t)),
            pl.BlockSpec((1, _T, C), lambda b, t: (b, t, 0)),
            pl.BlockSpec(memory_space=pltpu.SMEM),
        ],
        out_shape=[
            jax.ShapeDtypeStruct((B, _K, G), jnp.int32),
            jax.ShapeDtypeStruct((B, G, C), jnp.float32),
            jax.ShapeDtypeStruct((1,), jnp.float32),
        ],
    )(center, ctr_t, x_rest, x_rest, g1, b1)

    t1 = _sc_gather_sum(xn_tab.reshape(B * G, C),
                        idx.reshape(B * _K * G), B, G).reshape(B, G, C)

    sums = pl.pallas_call(
        _sums_kernel,
        grid=(B, nT),
        in_specs=[
            pl.BlockSpec((1, _T, C), lambda b, t: (b, t, 0)),
            pl.BlockSpec((1, _T, C), lambda b, t: (b, t, 0)),
        ],
        out_specs=pl.BlockSpec(memory_space=pltpu.SMEM),
        out_shape=jax.ShapeDtypeStruct((2,), jnp.float32),
    )(t1, xn_tab)

    M = B * G * _K * C
    var = (sums[1] + tqs[0] - sums[0] * sums[0] / M) / (M - 1)
    stdinv = (1.0 / (jnp.sqrt(var) + 1e-05)).reshape(1)

    out_rest, out_cls = pl.pallas_call(
        _mlp_kernel,
        grid=(B, nT),
        in_specs=[
            pl.BlockSpec((1, _T, C), lambda b, t: (b, t, 0)),
            pl.BlockSpec((1, _T, C), lambda b, t: (b, t, 0)),
            pl.BlockSpec((B, C), lambda b, t: (0, 0)),
            pl.BlockSpec(memory_space=pltpu.SMEM),
            pl.BlockSpec((1, C), lambda b, t: (0, 0)),
            pl.BlockSpec((1, C), lambda b, t: (0, 0)),
            pl.BlockSpec((1, C), lambda b, t: (0, 0)),
            pl.BlockSpec((1, C), lambda b, t: (0, 0)),
            pl.BlockSpec((1, 2 * C), lambda b, t: (0, 0)),
            pl.BlockSpec((1, 2 * C), lambda b, t: (0, 0)),
            pl.BlockSpec((2 * C, C), lambda b, t: (0, 0)),
            pl.BlockSpec((1, C), lambda b, t: (0, 0)),
            pl.BlockSpec((C, C), lambda b, t: (0, 0)),
            pl.BlockSpec((1, C), lambda b, t: (0, 0)),
            pl.BlockSpec((C, H), lambda b, t: (0, 0)),
            pl.BlockSpec((1, H), lambda b, t: (0, 0)),
            pl.BlockSpec((H, C), lambda b, t: (0, 0)),
            pl.BlockSpec((1, C), lambda b, t: (0, 0)),
        ],
        out_specs=[
            pl.BlockSpec((1, _T, C), lambda b, t: (b, t, 0)),
            pl.BlockSpec((B, C), lambda b, t: (0, 0)),
        ],
        out_shape=[
            jax.ShapeDtypeStruct((B, G, C), jnp.float32),
            jax.ShapeDtypeStruct((B, C), jnp.float32),
        ],
    )(t1, x_rest, x_cls, stdinv, g1, b1, g2, b2,
      affine_alpha.reshape(1, 2 * C), affine_beta.reshape(1, 2 * C),
      attn_w1, attn_b1.reshape(1, C), attn_w2, attn_b2.reshape(1, C),
      mlp_w1, mlp_b1.reshape(1, H), mlp_w2, mlp_b2.reshape(1, C))

    return jnp.concatenate([out_cls[:, None, :], out_rest], axis=1)
